# Initial kernel scaffold; baseline (speedup 1.0000x reference)
import jax
import jax.numpy as jnp
from jax.experimental import pallas as pl


def kernel(feat, sizes, W):
    raise NotImplementedError("write your pallas kernel here")


# speedup vs baseline: 8.6764x; 8.6764x over previous
import functools

import jax
import jax.numpy as jnp
from jax.experimental import pallas as pl
from jax.experimental.pallas import tpu as pltpu

WIN = 800


def _body(offs_ref, feat_hbm, w_ref, out_ref, buf, sems, *, n_rows, n_seg):
    g = pl.program_id(0)

    def window_base(seg):
        return jnp.minimum(offs_ref[seg], n_rows - WIN)

    def start_copy(seg, slot):
        pltpu.make_async_copy(
            feat_hbm.at[pl.ds(window_base(seg), WIN), :],
            buf.at[slot],
            sems.at[slot],
        ).start()

    @pl.when(g == 0)
    def _prologue():
        start_copy(0, 0)

    @pl.when(g + 1 < n_seg)
    def _prefetch_next():
        start_copy(g + 1, (g + 1) % 2)

    slot = g % 2
    pltpu.make_async_copy(
        feat_hbm.at[pl.ds(window_base(g), WIN), :],
        buf.at[slot],
        sems.at[slot],
    ).wait()

    start = offs_ref[g]
    end = offs_ref[g + 1]
    base = window_base(g)

    x = buf[slot]
    gidx = base + jax.lax.broadcasted_iota(jnp.int32, (WIN, 1), 0)
    mask = (gidx >= start) & (gidx < end)

    s = jax.lax.dot_general(
        x, w_ref[...], (((1,), (0,)), ((), ())),
        preferred_element_type=jnp.float32,
    )
    s = jnp.where(s >= 0, s, 0.2 * s)
    s = jnp.where(mask, s, -jnp.inf)
    m = jnp.max(s)
    m = jnp.where(jnp.isfinite(m), m, 0.0)
    p = jnp.where(mask, jnp.exp(s - m), 0.0)
    denom = jnp.maximum(jnp.sum(p), 1e-30)
    acc = jax.lax.dot_general(
        p, x, (((0,), (0,)), ((), ())),
        preferred_element_type=jnp.float32,
    )
    out_ref[...] = (acc / denom)[None]


def kernel(feat, sizes, W):
    n_rows, d = feat.shape
    b = sizes.shape[0]
    offsets = jnp.concatenate(
        [jnp.zeros((1,), jnp.int32), jnp.cumsum(sizes, dtype=jnp.int32)]
    )

    grid_spec = pltpu.PrefetchScalarGridSpec(
        num_scalar_prefetch=1,
        grid=(b,),
        in_specs=[
            pl.BlockSpec(memory_space=pl.ANY),
            pl.BlockSpec((d, 1), lambda g, offs: (0, 0)),
        ],
        out_specs=pl.BlockSpec((1, 1, d), lambda g, offs: (g, 0, 0)),
        scratch_shapes=[
            pltpu.VMEM((2, WIN, d), jnp.float32),
            pltpu.SemaphoreType.DMA((2,)),
        ],
    )

    out = pl.pallas_call(
        functools.partial(_body, n_rows=n_rows, n_seg=b),
        grid_spec=grid_spec,
        out_shape=jax.ShapeDtypeStruct((b, 1, d), jnp.float32),
    )(offsets, feat, W)
    return out.reshape(b, d)


# speedup vs baseline: 20.2739x; 2.3367x over previous
import functools

import jax
import jax.numpy as jnp
from jax import lax
from jax.experimental import pallas as pl
from jax.experimental.pallas import tpu as pltpu
from jax.experimental.pallas import tpu_sc as plsc

D = 128
L = 16
KD = D // L
C = 128
MAXCH = 7
BUFROWS = MAXCH * C
NEG_INF = float("-inf")

_GATHER_DNUMS = lax.GatherDimensionNumbers(
    offset_dims=(), collapsed_slice_dims=(0,), start_index_map=(0,))


def _take(v, idx):
    return lax.gather(v, idx[:, None], _GATHER_DNUMS, (1,),
                      mode=lax.GatherScatterMode.PROMISE_IN_BOUNDS)


def _tree_sum(v, lane):
    for sh in (1, 2, 4, 8):
        v = v + _take(v, lane ^ sh)
    return v


def _tree_max(v, lane):
    for sh in (1, 2, 4, 8):
        v = jnp.maximum(v, _take(v, lane ^ sh))
    return v


def _splat_lane(v, lane, j):
    return _take(v, (lane & 0) + j)


def _sc_body(n_rows, kpad, segs_per_w, nw, feat_hbm, st_hbm, sz_hbm, w_hbm,
             out_hbm, buf, sbuf, my_st, my_sz, w_v, orow, csems, ssem):
    cid = lax.axis_index("c")
    sid = lax.axis_index("s")
    wid = sid * 2 + cid

    pltpu.sync_copy(st_hbm.at[pl.ds(wid * kpad, kpad)], my_st.at[pl.ds(0, kpad)])
    pltpu.sync_copy(sz_hbm.at[pl.ds(wid * kpad, kpad)], my_sz.at[pl.ds(0, kpad)])
    pltpu.sync_copy(w_hbm, w_v)

    lane = lax.iota(jnp.int32, L)
    w_regs = [w_v[pl.ds(kk * L, L)] for kk in range(KD)]

    def do_segment(k, _):
        start = my_st[pl.ds(k, L)][0]
        size = my_sz[pl.ds(k, L)][0]
        end = start + size
        nch = lax.div(size + (C - 1), C)

        for ci in range(MAXCH):
            @pl.when(ci < nch)
            def _fire(ci=ci):
                base = jnp.minimum(start + ci * C, n_rows - C)
                pltpu.make_async_copy(
                    feat_hbm.at[pl.ds(base * D, C * D)],
                    buf.at[pl.ds(ci * C * D, C * D)],
                    csems.at[ci],
                ).start()

        def score_group(grp, carry):
            m_vec, ci_base, j0, j1 = carry
            q0 = ci_base + grp * L
            svec = jnp.zeros((L,), jnp.float32)
            for jj in range(L):
                q = q0 + jj
                acc = w_regs[0] * buf[pl.ds(q * D, L)]
                for kk in range(1, KD):
                    acc = acc + w_regs[kk] * buf[pl.ds(q * D + kk * L, L)]
                sv = _tree_sum(acc, lane)
                sv = jnp.maximum(sv, 0.2 * sv)
                svec = jnp.where(lane == jj, sv, svec)
            jloc = (q0 - ci_base) + lane
            valid = (jloc >= j0) & (jloc < j1)
            svec = jnp.where(valid, svec, NEG_INF)
            sbuf[pl.ds(q0, L)] = svec
            return (jnp.maximum(m_vec, svec), ci_base, j0, j1)

        m_vec = jnp.full((L,), NEG_INF, jnp.float32)
        for ci in range(MAXCH):
            @pl.when(ci < nch)
            def _wait(ci=ci):
                pltpu.make_async_copy(
                    feat_hbm.at[pl.ds(0, C * D)],
                    buf.at[pl.ds(ci * C * D, C * D)],
                    csems.at[ci],
                ).wait()

            s_i = start + ci * C
            base = jnp.minimum(s_i, n_rows - C)
            j0 = s_i - base
            j1 = jnp.minimum(s_i + C, end) - base
            live = ci < nch
            m_vec, _, _, _ = lax.fori_loop(
                0, jnp.where(live, C // L, 0), score_group,
                (m_vec, ci * C, j0, j1))

        m_splat = _tree_max(m_vec, lane)
        m_splat = jnp.where(size > 0, m_splat, jnp.zeros((L,), jnp.float32))
        ngrp = nch * (C // L)

        def expsum_group(g, dvec):
            pvec = jnp.exp(sbuf[pl.ds(g * L, L)] - m_splat)
            sbuf[pl.ds(g * L, L)] = pvec
            return dvec + pvec

        dvec = lax.fori_loop(0, ngrp, expsum_group,
                             jnp.zeros((L,), jnp.float32))
        inv_splat = 1.0 / jnp.maximum(_tree_sum(dvec, lane), 1e-30)

        def acc_group(g, accs):
            pvec = sbuf[pl.ds(g * L, L)]
            accs = list(accs)
            for jj in range(L):
                q = g * L + jj
                pb = _splat_lane(pvec, lane, jj)
                for kk in range(KD):
                    accs[kk] = accs[kk] + pb * buf[pl.ds(q * D + kk * L, L)]
            return tuple(accs)

        accs = tuple(jnp.zeros((L,), jnp.float32) for _ in range(KD))
        accs = lax.fori_loop(0, ngrp, acc_group, accs)

        for kk in range(KD):
            orow[pl.ds(kk * L, L)] = accs[kk] * inv_splat

        seg = wid + k * nw
        pltpu.sync_copy(orow, out_hbm.at[pl.ds(seg * D, D)])
        return 0

    lax.fori_loop(0, segs_per_w, do_segment, 0)


def kernel(feat, sizes, W):
    n_rows, d = feat.shape
    b = sizes.shape[0]
    nw = 32
    segs_per_w = (b + nw - 1) // nw
    kpad = ((segs_per_w + 31) // 32) * 32

    offsets = jnp.concatenate(
        [jnp.zeros((1,), jnp.int32), jnp.cumsum(sizes, dtype=jnp.int32)]
    )
    pad = kpad * nw - b
    starts = jnp.concatenate([offsets[:b], jnp.zeros((pad,), jnp.int32)])
    szs = jnp.concatenate([sizes, jnp.zeros((pad,), jnp.int32)])
    starts_w = starts.reshape(kpad, nw).T.reshape(-1)
    sizes_w = szs.reshape(kpad, nw).T.reshape(-1)
    w_flat = W.reshape(d)

    mesh = plsc.VectorSubcoreMesh(core_axis_name="c", subcore_axis_name="s")
    body = functools.partial(_sc_body, n_rows, kpad, segs_per_w, nw)
    run = pl.kernel(
        body,
        out_type=jax.ShapeDtypeStruct((b * d,), jnp.float32),
        mesh=mesh,
        scratch_types=[
            pltpu.VMEM((BUFROWS * d,), jnp.float32),
            pltpu.VMEM((BUFROWS,), jnp.float32),
            pltpu.VMEM((kpad + L,), jnp.int32),
            pltpu.VMEM((kpad + L,), jnp.int32),
            pltpu.VMEM((d,), jnp.float32),
            pltpu.VMEM((d,), jnp.float32),
            pltpu.SemaphoreType.DMA((MAXCH,)),
            pltpu.SemaphoreType.DMA,
        ],
    )
    out = run(feat.reshape(-1), starts_w, sizes_w, w_flat)
    return out.reshape(b, d)
